# Initial kernel scaffold; baseline (speedup 1.0000x reference)
import jax
import jax.numpy as jnp
from jax.experimental import pallas as pl


def kernel(x, W_route, b_route, W_noise, b_noise, fc1_w, fc1_b, fc2_w, fc2_b):
    raise NotImplementedError("write your pallas kernel here")


# speedup vs baseline: 2.4427x; 2.4427x over previous
import functools

import jax
import jax.numpy as jnp
from jax.experimental import pallas as pl
from jax.experimental.pallas import tpu as pltpu

B, S, D, H, E, K = 1, 2048, 1024, 4096, 8, 2
T = B * S
EP = 128


def _router_kernel(x_ref, w_ref, b_ref, eps_ref, gates_ref):
    x = x_ref[...]
    w = w_ref[...]
    res = jax.lax.dot_general(
        x.astype(jnp.bfloat16), w.astype(jnp.bfloat16),
        (((1,), (0,)), ((), ())),
        preferred_element_type=jnp.float32,
    )
    b = b_ref[0:1, :]
    logits = res[:, :EP] + b[:, :EP]
    nlogits = res[:, EP:] + b[:, EP:]
    sp = jnp.logaddexp(nlogits, 0.0)
    noisy = logits + eps_ref[...] * sp
    col = jax.lax.broadcasted_iota(jnp.int32, (T, EP), 1)
    neg = jnp.float32(-jnp.inf)
    noisy = jnp.where(col < E, noisy, neg)
    m1 = jnp.max(noisy, axis=1, keepdims=True)
    idx1 = jnp.min(jnp.where(noisy == m1, col, EP), axis=1, keepdims=True)
    v2 = jnp.where(col == idx1, neg, noisy)
    m2 = jnp.max(v2, axis=1, keepdims=True)
    idx2 = jnp.min(jnp.where(v2 == m2, col, EP), axis=1, keepdims=True)
    sel = (col == idx1) | (col == idx2)
    e1 = jnp.where(sel, jnp.exp(noisy - m1), 0.0)
    gates_ref[...] = e1 / jnp.sum(e1, axis=1, keepdims=True)


def _ffn_kernel(x_ref, w1_ref, b1_ref, w2_ref, b2_ref, g_ref, o_ref,
                acc_ref, *, n_hb):
    e = pl.program_id(0)
    hb = pl.program_id(1)

    @pl.when((e == 0) & (hb == 0))
    def _():
        acc_ref[...] = jnp.zeros_like(acc_ref)

    xb = x_ref[...].astype(jnp.bfloat16)
    w1 = w1_ref[0].astype(jnp.bfloat16)
    h = jax.lax.dot_general(
        xb, w1, (((1,), (0,)), ((), ())),
        preferred_element_type=jnp.float32)
    h = h + b1_ref[0]
    h = 0.5 * h * (1.0 + jax.lax.erf(h * 0.7071067811865476))
    w2 = w2_ref[0].astype(jnp.bfloat16)
    po = jax.lax.dot_general(
        h.astype(jnp.bfloat16), w2, (((1,), (0,)), ((), ())),
        preferred_element_type=jnp.float32)
    gcol = jax.lax.broadcasted_iota(jnp.int32, (T, EP), 1)
    ge = jnp.sum(jnp.where(gcol == e, g_ref[...], 0.0), axis=1, keepdims=True)
    acc_ref[...] += ge * po

    @pl.when(hb == 0)
    def _():
        acc_ref[...] += ge * b2_ref[0]

    @pl.when((e == E - 1) & (hb == n_hb - 1))
    def _():
        o_ref[...] = acc_ref[...]


def kernel(x, W_route, b_route, W_noise, b_noise, fc1_w, fc1_b, fc2_w, fc2_b):
    x2 = x.reshape(T, D)
    w = jnp.zeros((D, 2 * EP), jnp.float32)
    w = w.at[:, :E].set(W_route).at[:, EP:EP + E].set(W_noise)
    b = jnp.zeros((2 * EP,), jnp.float32)
    b = b.at[:E].set(b_route).at[EP:EP + E].set(b_noise)
    b = jnp.broadcast_to(b[None, :], (8, 2 * EP))
    eps = jax.random.normal(jax.random.key(42), (B, S, E), dtype=jnp.float32)
    eps_p = jnp.zeros((T, EP), jnp.float32).at[:, :E].set(eps.reshape(T, E))

    gates = pl.pallas_call(
        _router_kernel,
        out_shape=jax.ShapeDtypeStruct((T, EP), jnp.float32),
    )(x2, w, b, eps_p)

    n_hb = 4
    hbs = H // n_hb
    out = pl.pallas_call(
        functools.partial(_ffn_kernel, n_hb=n_hb),
        grid=(E, n_hb),
        in_specs=[
            pl.BlockSpec((T, D), lambda e, hb: (0, 0)),
            pl.BlockSpec((1, D, hbs), lambda e, hb: (e, 0, hb)),
            pl.BlockSpec((1, 1, hbs), lambda e, hb: (e, 0, hb)),
            pl.BlockSpec((1, hbs, D), lambda e, hb: (e, hb, 0)),
            pl.BlockSpec((1, 1, D), lambda e, hb: (e, 0, 0)),
            pl.BlockSpec((T, EP), lambda e, hb: (0, 0)),
        ],
        out_specs=pl.BlockSpec((T, D), lambda e, hb: (0, 0)),
        out_shape=jax.ShapeDtypeStruct((T, D), jnp.float32),
        scratch_shapes=[pltpu.VMEM((T, D), jnp.float32)],
    )(x2, fc1_w, fc1_b.reshape(E, 1, H), fc2_w, fc2_b.reshape(E, 1, D),
      gates)
    return out.reshape(B, S, D)
